# baseline (speedup 1.0000x reference)
import jax
import jax.numpy as jnp
from jax import lax
from jax.experimental import pallas as pl
from jax.experimental.pallas import tpu as pltpu

_EOS = 3
_NEG_INF = float("-inf")
_IMAX = 2**31 - 1


def _scan_kernel(sp_ref, probs_ref, cv_ref, ci_ref, buf, sem):
    j = pl.program_id(0)
    count = sp_ref[0]

    def copy(slot):
        r = sp_ref[1 + slot]
        return pltpu.make_async_copy(
            probs_ref.at[pl.ds(r, 1)], buf.at[pl.ds(lax.rem(slot, 2), 1)],
            sem.at[lax.rem(slot, 2)])

    @pl.when(jnp.logical_and(j == 0, count > 0))
    def _():
        copy(0).start()

    @pl.when(j + 1 < count)
    def _():
        copy(j + 1).start()

    @pl.when(j < count)
    def _():
        copy(j).wait()
        x = buf[lax.rem(j, 2)]
        V = x.shape[1]
        fi = lax.broadcasted_iota(jnp.int32, (1, V), 1)
        vals, idxs = [], []
        for _ in range(5):
            m = jnp.max(x)
            ci = jnp.min(jnp.where(x == m, fi, _IMAX))
            vals.append(m)
            idxs.append(ci)
            x = jnp.where(fi == ci, _NEG_INF, x)
        cv_ref[0, 0, :] = jnp.stack(vals)
        ci_ref[0, 0, :] = jnp.stack(idxs)


def _merge_kernel(cv_ref, ci_ref, lp_ref, pen_ref, len_ref,
                  sc_ref, pv_ref, ix_ref):
    cv = cv_ref[...]
    ci = ci_ref[...]
    lp = lp_ref[...]
    pen = pen_ref[...]
    done = len_ref[...] != 0
    B, K, _ = cv.shape
    V = 100000

    krow = lax.broadcasted_iota(jnp.int32, (B, K, 5), 1)
    x_act = jnp.where(done, _NEG_INF, (lp + cv) / pen)
    i_act = jnp.where(done, _IMAX, krow * V + ci)
    p_act = jnp.where(done, _NEG_INF, lp + cv)

    krow1 = lax.broadcasted_iota(jnp.int32, (B, K, 1), 1)
    x_eos = jnp.where(done, lp / pen, _NEG_INF)
    i_eos = jnp.where(done, krow1 * V + _EOS, _IMAX)
    p_eos = jnp.where(done, lp, _NEG_INF)

    X = jnp.concatenate([x_act, x_eos], axis=2)
    I = jnp.concatenate([i_act, i_eos], axis=2)
    P = jnp.concatenate([p_act, p_eos], axis=2)

    scs, pvs, ixs = [], [], []
    for _ in range(5):
        m = jnp.max(jnp.max(X, axis=2), axis=1)
        mb = m[:, None, None]
        c = jnp.min(jnp.min(jnp.where(X == mb, I, _IMAX), axis=2), axis=1)
        cb = c[:, None, None]
        pv = jnp.max(jnp.max(jnp.where(I == cb, P, _NEG_INF), axis=2), axis=1)
        scs.append(m)
        pvs.append(pv)
        ixs.append(c)
        X = jnp.where(I == cb, _NEG_INF, X)

    sc_ref[...] = jnp.stack(scs, axis=1)
    pv_ref[...] = jnp.stack(pvs, axis=1)
    ix_ref[...] = jnp.stack(ixs, axis=1)


def kernel(probs, log_probs, lengths, i, k):
    B, K, V = probs.shape
    R = B * K
    probs2 = probs.reshape(R, 1, V)
    len_flat = lengths.reshape(R)

    active = len_flat == 0
    order = jnp.argsort(jnp.logical_not(active), stable=True).astype(jnp.int32)
    count = jnp.sum(active).astype(jnp.int32)
    last = jnp.take(order, jnp.maximum(count - 1, 0))
    rows = jnp.where(jnp.arange(R, dtype=jnp.int32) < count, order, last)
    sp = jnp.concatenate([count[None], rows])

    cv, ci = pl.pallas_call(
        _scan_kernel,
        grid_spec=pltpu.PrefetchScalarGridSpec(
            num_scalar_prefetch=1,
            grid=(R,),
            in_specs=[
                pl.BlockSpec(memory_space=pl.ANY),
            ],
            out_specs=(
                pl.BlockSpec((1, 1, 5), lambda j, sp: (sp[j + 1], 0, 0)),
                pl.BlockSpec((1, 1, 5), lambda j, sp: (sp[j + 1], 0, 0)),
            ),
            scratch_shapes=[
                pltpu.VMEM((2, 1, V), jnp.float32),
                pltpu.SemaphoreType.DMA((2,)),
            ],
        ),
        out_shape=(
            jax.ShapeDtypeStruct((R, 1, 5), jnp.float32),
            jax.ShapeDtypeStruct((R, 1, 5), jnp.int32),
        ),
    )(sp, probs2)

    eff = jnp.where(lengths == 0, i + 1, lengths).astype(jnp.float32)
    pen = jnp.power((5.0 + eff) / 6.0, 0.8)

    full = lambda s: pl.BlockSpec(s, lambda: (0,) * len(s))
    sc, pv, ix = pl.pallas_call(
        _merge_kernel,
        in_specs=[
            full((B, K, 5)), full((B, K, 5)), full((B, K, 1)),
            full((B, K, 1)), full((B, K, 1)),
        ],
        out_specs=(full((B, 5)), full((B, 5)), full((B, 5))),
        out_shape=(
            jax.ShapeDtypeStruct((B, 5), jnp.float32),
            jax.ShapeDtypeStruct((B, 5), jnp.float32),
            jax.ShapeDtypeStruct((B, 5), jnp.int32),
        ),
    )(cv.reshape(B, K, 5), ci.reshape(B, K, 5), log_probs.reshape(B, K, 1),
      pen.reshape(B, K, 1), lengths.reshape(B, K, 1))

    best_idx = ix + jnp.asarray(k - K, jnp.int32)
    best_beams = best_idx // V
    best_tokens = best_idx % V
    return sc, pv, best_beams, best_tokens


# speedup vs baseline: 2.3231x; 2.3231x over previous
import jax
import jax.numpy as jnp
from jax import lax
from jax.experimental import pallas as pl
from jax.experimental.pallas import tpu as pltpu

_EOS = 3
_NEG_INF = float("-inf")
_IMAX = 2**31 - 1


_SL = 12500
_NC = 97
_TW = _SL - _NC * 128


def _scan_kernel(sp_ref, probs_ref, cv_ref, ci_ref, buf, sem):
    j = pl.program_id(0)
    count = sp_ref[0]

    def copy(slot):
        r = sp_ref[1 + slot]
        return pltpu.make_async_copy(
            probs_ref.at[pl.ds(r, 1)],
            buf.at[pl.ds(lax.rem(slot, 2), 1)],
            sem.at[lax.rem(slot, 2)])

    @pl.when(jnp.logical_and(j == 0, count > 0))
    def _():
        copy(0).start()

    @pl.when(j + 1 < count)
    def _():
        copy(j + 1).start()

    @pl.when(j < count)
    def _():
        copy(j).wait()
        xp = buf[lax.rem(j, 2)]
        sub = lax.broadcasted_iota(jnp.int32, (8, 128), 0)
        lane = lax.broadcasted_iota(jnp.int32, (8, 128), 1)

        acc = jnp.full((8, 128), _NEG_INF, jnp.float32)
        accid = jnp.zeros((8, 128), jnp.int32)
        for c in range(_NC):
            t = xp[:, c * 128:(c + 1) * 128]
            g = t > acc
            acc = jnp.where(g, t, acc)
            accid = jnp.where(g, c, accid)
        tail = xp[:, _NC * 128:_SL]
        gt = tail > acc[:, :_TW]
        acc = jnp.concatenate(
            [jnp.where(gt, tail, acc[:, :_TW]), acc[:, _TW:]], axis=1)
        accid = jnp.concatenate(
            [jnp.where(gt, _NC, accid[:, :_TW]), accid[:, _TW:]], axis=1)
        vidx = sub * _SL + accid * 128 + lane

        a2 = acc
        vals, idxs = [], []
        for _ in range(5):
            m = jnp.max(a2)
            ci = jnp.min(jnp.where(a2 == m, vidx, _IMAX))
            vals.append(m)
            idxs.append(ci)
            a2 = jnp.where(vidx == ci, _NEG_INF, a2)
        thr = vals[4]

        cacc = jnp.zeros((8, 128), jnp.int32)
        for c in range(_NC):
            cacc = cacc + (xp[:, c * 128:(c + 1) * 128] >= thr).astype(jnp.int32)
        cnt = jnp.sum(cacc) + jnp.sum((tail >= thr).astype(jnp.int32))

        @pl.when(cnt == 5)
        def _():
            cv_ref[0, 0, :] = jnp.stack(vals)
            ci_ref[0, 0, :] = jnp.stack(idxs)

        @pl.when(cnt != 5)
        def _():
            x = xp
            subf = lax.broadcasted_iota(jnp.int32, (8, _SL), 0)
            colf = lax.broadcasted_iota(jnp.int32, (8, _SL), 1)
            fi = subf * _SL + colf
            fvals, fidxs = [], []
            for _ in range(5):
                m = jnp.max(x)
                ci = jnp.min(jnp.where(x == m, fi, _IMAX))
                fvals.append(m)
                fidxs.append(ci)
                x = jnp.where(fi == ci, _NEG_INF, x)
            cv_ref[0, 0, :] = jnp.stack(fvals)
            ci_ref[0, 0, :] = jnp.stack(fidxs)


def _merge_kernel(cv_ref, ci_ref, lp_ref, pen_ref, len_ref,
                  sc_ref, pv_ref, ix_ref):
    cv = cv_ref[...]
    ci = ci_ref[...]
    lp = lp_ref[...]
    pen = pen_ref[...]
    done = len_ref[...] != 0
    B, K, _ = cv.shape
    V = 100000

    krow = lax.broadcasted_iota(jnp.int32, (B, K, 5), 1)
    x_act = jnp.where(done, _NEG_INF, (lp + cv) / pen)
    i_act = jnp.where(done, _IMAX, krow * V + ci)
    p_act = jnp.where(done, _NEG_INF, lp + cv)

    krow1 = lax.broadcasted_iota(jnp.int32, (B, K, 1), 1)
    x_eos = jnp.where(done, lp / pen, _NEG_INF)
    i_eos = jnp.where(done, krow1 * V + _EOS, _IMAX)
    p_eos = jnp.where(done, lp, _NEG_INF)

    X = jnp.concatenate([x_act, x_eos], axis=2)
    I = jnp.concatenate([i_act, i_eos], axis=2)
    P = jnp.concatenate([p_act, p_eos], axis=2)

    scs, pvs, ixs = [], [], []
    for _ in range(5):
        m = jnp.max(jnp.max(X, axis=2), axis=1)
        mb = m[:, None, None]
        c = jnp.min(jnp.min(jnp.where(X == mb, I, _IMAX), axis=2), axis=1)
        cb = c[:, None, None]
        pv = jnp.max(jnp.max(jnp.where(I == cb, P, _NEG_INF), axis=2), axis=1)
        scs.append(m)
        pvs.append(pv)
        ixs.append(c)
        X = jnp.where(I == cb, _NEG_INF, X)

    sc_ref[...] = jnp.stack(scs, axis=1)
    pv_ref[...] = jnp.stack(pvs, axis=1)
    ix_ref[...] = jnp.stack(ixs, axis=1)


def kernel(probs, log_probs, lengths, i, k):
    B, K, V = probs.shape
    R = B * K
    probs2 = probs.reshape(R, 8, _SL)
    len_flat = lengths.reshape(R)

    active = len_flat == 0
    order = jnp.argsort(jnp.logical_not(active), stable=True).astype(jnp.int32)
    count = jnp.sum(active).astype(jnp.int32)
    last = jnp.take(order, jnp.maximum(count - 1, 0))
    rows = jnp.where(jnp.arange(R, dtype=jnp.int32) < count, order, last)
    sp = jnp.concatenate([count[None], rows])

    cv, ci = pl.pallas_call(
        _scan_kernel,
        grid_spec=pltpu.PrefetchScalarGridSpec(
            num_scalar_prefetch=1,
            grid=(R,),
            in_specs=[
                pl.BlockSpec(memory_space=pl.ANY),
            ],
            out_specs=(
                pl.BlockSpec((1, 1, 5), lambda j, sp: (sp[j + 1], 0, 0)),
                pl.BlockSpec((1, 1, 5), lambda j, sp: (sp[j + 1], 0, 0)),
            ),
            scratch_shapes=[
                pltpu.VMEM((2, 8, _SL), jnp.float32),
                pltpu.SemaphoreType.DMA((2,)),
            ],
        ),
        out_shape=(
            jax.ShapeDtypeStruct((R, 1, 5), jnp.float32),
            jax.ShapeDtypeStruct((R, 1, 5), jnp.int32),
        ),
    )(sp, probs2)

    eff = jnp.where(lengths == 0, i + 1, lengths).astype(jnp.float32)
    pen = jnp.power((5.0 + eff) / 6.0, 0.8)

    full = lambda s: pl.BlockSpec(s, lambda: (0,) * len(s))
    sc, pv, ix = pl.pallas_call(
        _merge_kernel,
        in_specs=[
            full((B, K, 5)), full((B, K, 5)), full((B, K, 1)),
            full((B, K, 1)), full((B, K, 1)),
        ],
        out_specs=(full((B, 5)), full((B, 5)), full((B, 5))),
        out_shape=(
            jax.ShapeDtypeStruct((B, 5), jnp.float32),
            jax.ShapeDtypeStruct((B, 5), jnp.float32),
            jax.ShapeDtypeStruct((B, 5), jnp.int32),
        ),
    )(cv.reshape(B, K, 5), ci.reshape(B, K, 5), log_probs.reshape(B, K, 1),
      pen.reshape(B, K, 1), lengths.reshape(B, K, 1))

    best_idx = ix + jnp.asarray(k - K, jnp.int32)
    best_beams = best_idx // V
    best_tokens = best_idx % V
    return sc, pv, best_beams, best_tokens


# speedup vs baseline: 3.0804x; 1.3260x over previous
import jax
import jax.numpy as jnp
from jax import lax
from jax.experimental import pallas as pl
from jax.experimental.pallas import tpu as pltpu

_EOS = 3
_NEG_INF = float("-inf")
_IMAX = 2**31 - 1

_V = 100000
_NC = _V // 128
_TW = _V - _NC * 128
_NA = 8


def _scan_kernel(sp_ref, probs_ref, cv_ref, ci_ref, buf, sem):
    j = pl.program_id(0)
    count = sp_ref[0]

    def copy(slot):
        b = sp_ref[1 + slot]
        return pltpu.make_async_copy(
            probs_ref.at[pl.ds(b, 1)], buf.at[pl.ds(lax.rem(slot, 2), 1)],
            sem.at[lax.rem(slot, 2)])

    @pl.when(jnp.logical_and(j == 0, count > 0))
    def _():
        copy(0).start()

    @pl.when(j + 1 < count)
    def _():
        copy(j + 1).start()

    @pl.when(j < count)
    def _():
        copy(j).wait()
        xp = buf[lax.rem(j, 2)]
        K = xp.shape[0]
        lane = lax.broadcasted_iota(jnp.int32, (K, 128), 1)

        accs = [jnp.full((K, 128), _NEG_INF, jnp.float32) for _ in range(_NA)]
        ids = [jnp.zeros((K, 128), jnp.int32) for _ in range(_NA)]
        for c in range(_NC):
            a = c % _NA
            t = xp[:, c * 128:(c + 1) * 128]
            g = t > accs[a]
            accs[a] = jnp.where(g, t, accs[a])
            ids[a] = jnp.where(g, c, ids[a])
        tail = xp[:, _NC * 128:_V]
        gt = tail > accs[0][:, :_TW]
        accs[0] = jnp.concatenate(
            [jnp.where(gt, tail, accs[0][:, :_TW]), accs[0][:, _TW:]], axis=1)
        ids[0] = jnp.concatenate(
            [jnp.where(gt, _NC, ids[0][:, :_TW]), ids[0][:, _TW:]], axis=1)

        acc = jnp.concatenate(accs, axis=1)
        accid = jnp.concatenate(ids, axis=1)
        lane8 = jnp.concatenate([lane] * _NA, axis=1)
        vidx = accid * 128 + lane8

        a2 = acc
        vals, idxs = [], []
        for _ in range(5):
            m = jnp.max(a2, axis=1, keepdims=True)
            ci = jnp.min(jnp.where(a2 == m, vidx, _IMAX),
                         axis=1, keepdims=True)
            vals.append(m)
            idxs.append(ci)
            a2 = jnp.where(vidx == ci, _NEG_INF, a2)
        thr = vals[4]

        cacc = jnp.zeros((K, 128), jnp.int32)
        for c in range(_NC):
            cacc = cacc + (xp[:, c * 128:(c + 1) * 128] >= thr).astype(jnp.int32)
        cnt = (jnp.sum(cacc, axis=1, keepdims=True)
               + jnp.sum((tail >= thr).astype(jnp.int32), axis=1, keepdims=True))
        allok = jnp.all(cnt == 5)

        @pl.when(allok)
        def _():
            cv_ref[0] = jnp.concatenate(vals, axis=1)
            ci_ref[0] = jnp.concatenate(idxs, axis=1)

        @pl.when(jnp.logical_not(allok))
        def _():
            x = xp
            fi = lax.broadcasted_iota(jnp.int32, (K, _V), 1)
            fvals, fidxs = [], []
            for _ in range(5):
                m = jnp.max(x, axis=1, keepdims=True)
                ci = jnp.min(jnp.where(x == m, fi, _IMAX),
                             axis=1, keepdims=True)
                fvals.append(m)
                fidxs.append(ci)
                x = jnp.where(fi == ci, _NEG_INF, x)
            cv_ref[0] = jnp.concatenate(fvals, axis=1)
            ci_ref[0] = jnp.concatenate(fidxs, axis=1)


def _merge_kernel(cv_ref, ci_ref, lp_ref, pen_ref, len_ref,
                  sc_ref, pv_ref, ix_ref):
    cv = cv_ref[...]
    ci = ci_ref[...]
    lp = lp_ref[...]
    pen = pen_ref[...]
    done = len_ref[...] != 0
    B, K, _ = cv.shape

    krow = lax.broadcasted_iota(jnp.int32, (B, K, 5), 1)
    x_act = jnp.where(done, _NEG_INF, (lp + cv) / pen)
    i_act = jnp.where(done, _IMAX, krow * _V + ci)
    p_act = jnp.where(done, _NEG_INF, lp + cv)

    krow1 = lax.broadcasted_iota(jnp.int32, (B, K, 1), 1)
    x_eos = jnp.where(done, lp / pen, _NEG_INF)
    i_eos = jnp.where(done, krow1 * _V + _EOS, _IMAX)
    p_eos = jnp.where(done, lp, _NEG_INF)

    X = jnp.concatenate([x_act, x_eos], axis=2)
    I = jnp.concatenate([i_act, i_eos], axis=2)
    P = jnp.concatenate([p_act, p_eos], axis=2)

    scs, pvs, ixs = [], [], []
    for _ in range(5):
        m = jnp.max(jnp.max(X, axis=2), axis=1)
        mb = m[:, None, None]
        c = jnp.min(jnp.min(jnp.where(X == mb, I, _IMAX), axis=2), axis=1)
        cb = c[:, None, None]
        pv = jnp.max(jnp.max(jnp.where(I == cb, P, _NEG_INF), axis=2), axis=1)
        scs.append(m)
        pvs.append(pv)
        ixs.append(c)
        X = jnp.where(I == cb, _NEG_INF, X)

    sc_ref[...] = jnp.stack(scs, axis=1)
    pv_ref[...] = jnp.stack(pvs, axis=1)
    ix_ref[...] = jnp.stack(ixs, axis=1)


def kernel(probs, log_probs, lengths, i, k):
    B, K, V = probs.shape

    batch_active = jnp.any(lengths == 0, axis=1)
    order = jnp.argsort(jnp.logical_not(batch_active),
                        stable=True).astype(jnp.int32)
    count = jnp.sum(batch_active).astype(jnp.int32)
    last = jnp.take(order, jnp.maximum(count - 1, 0))
    rows = jnp.where(jnp.arange(B, dtype=jnp.int32) < count, order, last)
    sp = jnp.concatenate([count[None], rows])

    cv, ci = pl.pallas_call(
        _scan_kernel,
        grid_spec=pltpu.PrefetchScalarGridSpec(
            num_scalar_prefetch=1,
            grid=(B,),
            in_specs=[
                pl.BlockSpec(memory_space=pl.ANY),
            ],
            out_specs=(
                pl.BlockSpec((1, K, 5), lambda j, sp: (sp[j + 1], 0, 0)),
                pl.BlockSpec((1, K, 5), lambda j, sp: (sp[j + 1], 0, 0)),
            ),
            scratch_shapes=[
                pltpu.VMEM((2, K, V), jnp.float32),
                pltpu.SemaphoreType.DMA((2,)),
            ],
        ),
        out_shape=(
            jax.ShapeDtypeStruct((B, K, 5), jnp.float32),
            jax.ShapeDtypeStruct((B, K, 5), jnp.int32),
        ),
    )(sp, probs)

    eff = jnp.where(lengths == 0, i + 1, lengths).astype(jnp.float32)
    pen = jnp.power((5.0 + eff) / 6.0, 0.8)

    full = lambda s: pl.BlockSpec(s, lambda: (0,) * len(s))
    sc, pv, ix = pl.pallas_call(
        _merge_kernel,
        in_specs=[
            full((B, K, 5)), full((B, K, 5)), full((B, K, 1)),
            full((B, K, 1)), full((B, K, 1)),
        ],
        out_specs=(full((B, 5)), full((B, 5)), full((B, 5))),
        out_shape=(
            jax.ShapeDtypeStruct((B, 5), jnp.float32),
            jax.ShapeDtypeStruct((B, 5), jnp.float32),
            jax.ShapeDtypeStruct((B, 5), jnp.int32),
        ),
    )(cv, ci, log_probs.reshape(B, K, 1),
      pen.reshape(B, K, 1), lengths.reshape(B, K, 1))

    best_idx = ix + jnp.asarray(k - K, jnp.int32)
    best_beams = best_idx // V
    best_tokens = best_idx % V
    return sc, pv, best_beams, best_tokens
